# Initial kernel scaffold; baseline (speedup 1.0000x reference)
import jax
import jax.numpy as jnp
from jax.experimental import pallas as pl


def kernel(visit_concept_orders, pe):
    raise NotImplementedError("write your pallas kernel here")


# speedup vs baseline: 5.0866x; 5.0866x over previous
import functools

import jax
import jax.numpy as jnp
from jax import lax
from jax.experimental import pallas as pl
from jax.experimental.pallas import tpu as pltpu
from jax.experimental.pallas import tpu_sc as plsc

B = 16384
S = 200
D = 128
NC = 2
NS = 16
NW = NC * NS
ROWS_PER_W = B // NW

_MIN_OFFS = tuple(range(0, 192, 16)) + (184,)
_HALF_OFFS = tuple(range(0, 96, 16)) + (84,)


def _body(vco_hbm, pe_hbm, out_hbm, idx_v, shifted_v, rows_v, sem):
    wid = lax.axis_index("s") * NC + lax.axis_index("c")
    base = wid * ROWS_PER_W

    def row_step(i, carry):
        r = base + i
        pltpu.sync_copy(vco_hbm.at[r], idx_v)

        acc = idx_v[pl.ds(0, 16)]
        for off in _MIN_OFFS[1:]:
            acc = jnp.minimum(acc, idx_v[pl.ds(off, 16)])
        m = acc[0]
        for j in range(1, 16):
            m = jnp.minimum(m, acc[j])

        for h in range(2):
            for off in _HALF_OFFS:
                shifted_v[h, pl.ds(off, 16)] = idx_v[pl.ds(h * 100 + off, 16)] - m

        cp0 = pltpu.async_copy(pe_hbm.at[shifted_v.at[0]], rows_v.at[pl.ds(0, 100)], sem)
        cp1 = pltpu.async_copy(pe_hbm.at[shifted_v.at[1]], rows_v.at[pl.ds(100, 100)], sem)
        cp0.wait()
        cp1.wait()
        pltpu.sync_copy(rows_v, out_hbm.at[r])
        return carry

    lax.fori_loop(0, ROWS_PER_W, row_step, 0)


@functools.partial(
    pl.kernel,
    out_type=jax.ShapeDtypeStruct((B, S, D), jnp.float32),
    mesh=plsc.VectorSubcoreMesh(core_axis_name="c", subcore_axis_name="s"),
    scratch_types=[
        pltpu.VMEM((S,), jnp.int32),
        pltpu.VMEM((2, 100), jnp.int32),
        pltpu.VMEM((S, D), jnp.float32),
        pltpu.SemaphoreType.DMA,
    ],
)
def _pe_lookup(vco_hbm, pe_hbm, out_hbm, idx_v, shifted_v, rows_v, sem):
    _body(vco_hbm, pe_hbm, out_hbm, idx_v, shifted_v, rows_v, sem)


def kernel(visit_concept_orders, pe):
    return _pe_lookup(visit_concept_orders, pe)


# speedup vs baseline: 5.1620x; 1.0148x over previous
import functools

import jax
import jax.numpy as jnp
from jax import lax
from jax.experimental import pallas as pl
from jax.experimental.pallas import tpu as pltpu
from jax.experimental.pallas import tpu_sc as plsc

B = 16384
S = 200
D = 128
NC = 2
NS = 16
NW = NC * NS
ROWS_PER_W = B // NW
CH = 8
NCH = ROWS_PER_W // CH

_MIN_OFFS = tuple(range(0, 192, 16)) + (184,)
_HALF_OFFS = tuple(range(0, 96, 16)) + (84,)


def _body(vco_hbm, pe_hbm, out_hbm, idxc, shifted_v, rows0, rows1, gsem, osem0, osem1):
    wid = lax.axis_index("s") * NC + lax.axis_index("c")
    base = wid * ROWS_PER_W

    def chunk_step(k, carry):
        r0 = base + k * CH
        pltpu.sync_copy(vco_hbm.at[pl.ds(r0, CH)], idxc)
        for j in range(CH):
            r = r0 + j
            rows_v, osem = (rows0, osem0) if j % 2 == 0 else (rows1, osem1)

            acc = idxc[j, pl.ds(0, 16)]
            for off in _MIN_OFFS[1:]:
                acc = jnp.minimum(acc, idxc[j, pl.ds(off, 16)])
            m = acc[0]
            for t in range(1, 16):
                m = jnp.minimum(m, acc[t])

            for h in range(2):
                for off in _HALF_OFFS:
                    shifted_v[h, pl.ds(off, 16)] = idxc[j, pl.ds(h * 100 + off, 16)] - m

            if j >= 2:
                pltpu.make_async_copy(rows_v, out_hbm.at[r], osem).wait()
            else:
                @pl.when(k > 0)
                def _wait_prev():
                    pltpu.make_async_copy(rows_v, out_hbm.at[r], osem).wait()

            g0 = pltpu.async_copy(pe_hbm.at[shifted_v.at[0]], rows_v.at[pl.ds(0, 100)], gsem)
            g1 = pltpu.async_copy(pe_hbm.at[shifted_v.at[1]], rows_v.at[pl.ds(100, 100)], gsem)
            g0.wait()
            g1.wait()
            pltpu.async_copy(rows_v, out_hbm.at[r], osem)
        return carry

    lax.fori_loop(0, NCH, chunk_step, 0)
    pltpu.make_async_copy(rows0, out_hbm.at[base], osem0).wait()
    pltpu.make_async_copy(rows1, out_hbm.at[base], osem1).wait()


@functools.partial(
    pl.kernel,
    out_type=jax.ShapeDtypeStruct((B, S, D), jnp.float32),
    mesh=plsc.VectorSubcoreMesh(core_axis_name="c", subcore_axis_name="s"),
    scratch_types=[
        pltpu.VMEM((CH, S), jnp.int32),
        pltpu.VMEM((2, 100), jnp.int32),
        pltpu.VMEM((S, D), jnp.float32),
        pltpu.VMEM((S, D), jnp.float32),
        pltpu.SemaphoreType.DMA,
        pltpu.SemaphoreType.DMA,
        pltpu.SemaphoreType.DMA,
    ],
)
def _pe_lookup(vco_hbm, pe_hbm, out_hbm, idxc, shifted_v, rows0, rows1, gsem, osem0, osem1):
    _body(vco_hbm, pe_hbm, out_hbm, idxc, shifted_v, rows0, rows1, gsem, osem0, osem1)


def kernel(visit_concept_orders, pe):
    return _pe_lookup(visit_concept_orders, pe)
